# baseline (speedup 1.0000x reference)
import functools

import jax
import jax.numpy as jnp
from jax import lax
from jax.experimental import pallas as pl
from jax.experimental.pallas import tpu as pltpu
from jax.experimental.pallas import tpu_sc as plsc

N_CLASSES = 2
EPS = 1e-08

B = 8
NSC = 2
NTC = B - NSC
H = 512
W = 512
NC = 2
NS = 16
NW = NC * NS
ROWS = H // NW
LANES = 16

_mesh = plsc.VectorSubcoreMesh(core_axis_name="c", subcore_axis_name="s")


@functools.partial(
    pl.kernel,
    mesh=_mesh,
    out_type=jax.ShapeDtypeStruct((NW, 3 * NSC, LANES), jnp.int32),
    scratch_types=[
        pltpu.VMEM((2, ROWS, W), jnp.float32),
        pltpu.VMEM((2, ROWS, W), jnp.int32),
        pltpu.VMEM((3 * NSC, LANES), jnp.int32),
        pltpu.SemaphoreType.DMA,
        pltpu.SemaphoreType.DMA,
        pltpu.SemaphoreType.DMA,
        pltpu.SemaphoreType.DMA,
    ],
)
def _stage1(pr_hbm, gt_hbm, part_hbm, pr_buf, gt_buf, part_buf,
            sem_pr0, sem_pr1, sem_gt0, sem_gt1):
    wid = lax.axis_index("s") * NC + lax.axis_index("c")
    pr_sems = (sem_pr0, sem_pr1)
    gt_sems = (sem_gt0, sem_gt1)

    copies = {}

    row0 = wid * ROWS

    def issue(s):
        b = s % 2
        copies[b] = (
            pltpu.async_copy(pr_hbm.at[s, 0, pl.ds(row0, ROWS), :],
                             pr_buf.at[b], pr_sems[b]),
            pltpu.async_copy(gt_hbm.at[s, pl.ds(row0, ROWS), :],
                             gt_buf.at[b], gt_sems[b]),
        )

    issue(0)

    zeros = jnp.zeros((LANES,), jnp.int32)
    NACC = 4

    for s in range(NSC):
        b = s % 2
        c_pr, c_gt = copies[b]
        c_pr.wait()
        c_gt.wait()
        if s + 1 < NSC:
            issue(s + 1)

        def body(i, carry, b=b):
            tp, p, g = list(carry[0]), list(carry[1]), list(carry[2])
            off = i * LANES
            for r in range(ROWS):
                a = r % NACC
                pr = pr_buf[b, r, pl.ds(off, LANES)]
                gt = gt_buf[b, r, pl.ds(off, LANES)]
                pred = pr > 0.0
                tp[a] = tp[a] + jnp.where(pred, gt, 0)
                p[a] = p[a] + jnp.where(pred, 1, 0)
                g[a] = g[a] + gt
            return tuple(tp), tuple(p), tuple(g)

        init = tuple(tuple(zeros for _ in range(NACC)) for _ in range(3))
        tp, p, g = lax.fori_loop(0, W // LANES, body, init)
        part_buf[s, :] = tp[0] + tp[1] + (tp[2] + tp[3])
        part_buf[NSC + s, :] = p[0] + p[1] + (p[2] + p[3])
        part_buf[2 * NSC + s, :] = g[0] + g[1] + (g[2] + g[3])

    pltpu.sync_copy(part_buf, part_hbm.at[wid])


HSPLIT = 4


def _tc_reduce(pr_ref, gt_ref, out_ref):
    j = pl.program_id(1)
    pred = (pr_ref[0, 0] > 0.0).astype(jnp.float32)
    gt_f = gt_ref[0].astype(jnp.float32)
    tp = jnp.sum(pred * gt_f)
    p = jnp.sum(pred)
    g = jnp.sum(gt_f)
    li = lax.broadcasted_iota(jnp.int32, (1, 1, 128), 2)
    vals = jnp.where(li == 0, tp,
                     jnp.where(li == 1, p,
                               jnp.where(li == 2, g, 0.0)))

    @pl.when(j == 0)
    def _():
        out_ref[...] = vals

    @pl.when(j != 0)
    def _():
        out_ref[...] = out_ref[...] + vals


def _combine(sc_ref, tc_ref, out_ref):
    x = sc_ref[...].astype(jnp.float32)
    r = jnp.sum(x, axis=0)
    tp_sc = jnp.sum(r[0:NSC, :], axis=1)
    p_sc = jnp.sum(r[NSC:2 * NSC, :], axis=1)
    g_sc = jnp.sum(r[2 * NSC:3 * NSC, :], axis=1)
    t = tc_ref[...]
    pm = rm = fm = im = 0.0
    for s in range(B):
        if s < NSC:
            tp, p, g = tp_sc[s], p_sc[s], g_sc[s]
        else:
            tp, p, g = (t[s - NSC, 0, 0], t[s - NSC, 0, 1],
                        t[s - NSC, 0, 2])
        precision = (tp + EPS) / (p + EPS)
        recall = (tp + EPS) / (g + EPS)
        f1 = 2.0 * precision * recall / (precision + recall)
        iou = (tp + EPS) / (p + g - tp + EPS)
        pm += precision
        rm += recall
        fm += f1
        im += iou
    inv_b = 1.0 / B
    li = lax.broadcasted_iota(jnp.int32, (4,), 0)
    out_ref[...] = jnp.where(li == 0, pm * inv_b,
                             jnp.where(li == 1, rm * inv_b,
                                       jnp.where(li == 2, fm * inv_b,
                                                 im * inv_b)))


def kernel(y_pr, y_gt):
    gt = y_gt.astype(jnp.int32)
    sc_partials = _stage1(y_pr, gt)
    tc_partials = pl.pallas_call(
        _tc_reduce,
        grid=(NTC, HSPLIT),
        in_specs=[
            pl.BlockSpec((1, 1, H // HSPLIT, W),
                         lambda i, j: (NSC + i, 0, j, 0)),
            pl.BlockSpec((1, H // HSPLIT, W),
                         lambda i, j: (NSC + i, j, 0)),
        ],
        out_specs=pl.BlockSpec((1, 1, 128), lambda i, j: (i, 0, 0)),
        out_shape=jax.ShapeDtypeStruct((NTC, 1, 128), jnp.float32),
    )(y_pr, gt)
    return pl.pallas_call(
        _combine,
        out_shape=jax.ShapeDtypeStruct((4,), jnp.float32),
    )(sc_partials, tc_partials)


# speedup vs baseline: 1.3364x; 1.3364x over previous
import functools

import jax
import jax.numpy as jnp
from jax import lax
from jax.experimental import pallas as pl
from jax.experimental.pallas import tpu as pltpu
from jax.experimental.pallas import tpu_sc as plsc

N_CLASSES = 2
EPS = 1e-08

B = 8
NSC = 2
NTC = B - NSC
H = 512
W = 512
NC = 2
NS = 16
NW = NC * NS
ROWS = H // NW
LANES = 16

_mesh = plsc.VectorSubcoreMesh(core_axis_name="c", subcore_axis_name="s")


@functools.partial(
    pl.kernel,
    mesh=_mesh,
    out_type=jax.ShapeDtypeStruct((NW, 3 * NSC, LANES), jnp.int32),
    scratch_types=[
        pltpu.VMEM((2, ROWS, W), jnp.float32),
        pltpu.VMEM((2, ROWS, W), jnp.int32),
        pltpu.VMEM((3 * NSC, LANES), jnp.int32),
        pltpu.SemaphoreType.DMA,
        pltpu.SemaphoreType.DMA,
        pltpu.SemaphoreType.DMA,
        pltpu.SemaphoreType.DMA,
    ],
)
def _stage1(pr_hbm, gt_hbm, part_hbm, pr_buf, gt_buf, part_buf,
            sem_pr0, sem_pr1, sem_gt0, sem_gt1):
    wid = lax.axis_index("s") * NC + lax.axis_index("c")
    pr_sems = (sem_pr0, sem_pr1)
    gt_sems = (sem_gt0, sem_gt1)

    copies = {}

    row0 = wid * ROWS

    def issue(s):
        b = s % 2
        copies[b] = (
            pltpu.async_copy(pr_hbm.at[s, 0, pl.ds(row0, ROWS), :],
                             pr_buf.at[b], pr_sems[b]),
            pltpu.async_copy(gt_hbm.at[s, pl.ds(row0, ROWS), :],
                             gt_buf.at[b], gt_sems[b]),
        )

    issue(0)

    zeros = jnp.zeros((LANES,), jnp.int32)
    NACC = 4

    for s in range(NSC):
        b = s % 2
        c_pr, c_gt = copies[b]
        c_pr.wait()
        c_gt.wait()
        if s + 1 < NSC:
            issue(s + 1)

        def body(i, carry, b=b):
            tp, p, g = list(carry[0]), list(carry[1]), list(carry[2])
            off = i * LANES
            for r in range(ROWS):
                a = r % NACC
                pr = pr_buf[b, r, pl.ds(off, LANES)]
                gt = gt_buf[b, r, pl.ds(off, LANES)]
                pred = pr > 0.0
                tp[a] = tp[a] + jnp.where(pred, gt, 0)
                p[a] = p[a] + jnp.where(pred, 1, 0)
                g[a] = g[a] + gt
            return tuple(tp), tuple(p), tuple(g)

        init = tuple(tuple(zeros for _ in range(NACC)) for _ in range(3))
        tp, p, g = lax.fori_loop(0, W // LANES, body, init)
        part_buf[s, :] = tp[0] + tp[1] + (tp[2] + tp[3])
        part_buf[NSC + s, :] = p[0] + p[1] + (p[2] + p[3])
        part_buf[2 * NSC + s, :] = g[0] + g[1] + (g[2] + g[3])

    pltpu.sync_copy(part_buf, part_hbm.at[wid])


HSPLIT = 1


def _tc_reduce(pr_ref, gt_ref, out_ref):
    j = pl.program_id(1)
    pred = (pr_ref[0, 0] > 0.0).astype(jnp.float32)
    gt_f = gt_ref[0].astype(jnp.float32)
    tp = jnp.sum(pred * gt_f)
    p = jnp.sum(pred)
    g = jnp.sum(gt_f)
    li = lax.broadcasted_iota(jnp.int32, (1, 1, 128), 2)
    vals = jnp.where(li == 0, tp,
                     jnp.where(li == 1, p,
                               jnp.where(li == 2, g, 0.0)))

    @pl.when(j == 0)
    def _():
        out_ref[...] = vals

    @pl.when(j != 0)
    def _():
        out_ref[...] = out_ref[...] + vals


def _combine(sc_ref, tc_ref, out_ref):
    x = sc_ref[...].astype(jnp.float32)
    r = jnp.sum(x, axis=0)
    tp_sc = jnp.sum(r[0:NSC, :], axis=1)
    p_sc = jnp.sum(r[NSC:2 * NSC, :], axis=1)
    g_sc = jnp.sum(r[2 * NSC:3 * NSC, :], axis=1)
    t = tc_ref[...]
    pm = rm = fm = im = 0.0
    for s in range(B):
        if s < NSC:
            tp, p, g = tp_sc[s], p_sc[s], g_sc[s]
        else:
            tp, p, g = (t[s - NSC, 0, 0], t[s - NSC, 0, 1],
                        t[s - NSC, 0, 2])
        precision = (tp + EPS) / (p + EPS)
        recall = (tp + EPS) / (g + EPS)
        f1 = 2.0 * precision * recall / (precision + recall)
        iou = (tp + EPS) / (p + g - tp + EPS)
        pm += precision
        rm += recall
        fm += f1
        im += iou
    inv_b = 1.0 / B
    li = lax.broadcasted_iota(jnp.int32, (4,), 0)
    out_ref[...] = jnp.where(li == 0, pm * inv_b,
                             jnp.where(li == 1, rm * inv_b,
                                       jnp.where(li == 2, fm * inv_b,
                                                 im * inv_b)))


def kernel(y_pr, y_gt):
    gt = y_gt.astype(jnp.int32)
    sc_partials = _stage1(y_pr, gt)
    tc_partials = pl.pallas_call(
        _tc_reduce,
        grid=(NTC, HSPLIT),
        in_specs=[
            pl.BlockSpec((1, 1, H // HSPLIT, W),
                         lambda i, j: (NSC + i, 0, j, 0)),
            pl.BlockSpec((1, H // HSPLIT, W),
                         lambda i, j: (NSC + i, j, 0)),
        ],
        out_specs=pl.BlockSpec((1, 1, 128), lambda i, j: (i, 0, 0)),
        out_shape=jax.ShapeDtypeStruct((NTC, 1, 128), jnp.float32),
    )(y_pr, gt)
    return pl.pallas_call(
        _combine,
        out_shape=jax.ShapeDtypeStruct((4,), jnp.float32),
    )(sc_partials, tc_partials)
